# baseline (speedup 1.0000x reference)
import functools

import jax
import jax.numpy as jnp
from jax import lax
from jax.experimental import pallas as pl
from jax.experimental.pallas import tpu as pltpu
from jax.experimental.pallas import tpu_sc as plsc

_NC = 2
_NS = 16
_L = 16
_TR, _TC_ = 8, 128


def _sc_pos_kernel(h, w, d, b, row_hbm, col_hbm, out_hbm, col_v, rowg_v, buf_v, sem, insem):
    wid = lax.axis_index("s") * _NC + lax.axis_index("c")

    in_handles = []
    for g in range(w // _TR):
        for c in range(d // _TC_):
            in_handles.append(
                pltpu.async_copy(
                    col_hbm.at[pl.ds(g * _TR, _TR), pl.ds(c * _TC_, _TC_)],
                    col_v.at[pl.ds(g * _TR, _TR), pl.ds(c * _TC_, _TC_)],
                    insem,
                )
            )
    grp = (wid // _TR) * _TR
    for c in range(d // _TC_):
        in_handles.append(
            pltpu.async_copy(
                row_hbm.at[pl.ds(grp, _TR), pl.ds(c * _TC_, _TC_)],
                rowg_v.at[:, pl.ds(c * _TC_, _TC_)],
                insem,
            )
        )
    for hnd in in_handles:
        hnd.wait()
    rsub = wid % _TR
    rvecs = [rowg_v[rsub, pl.ds(k * _L, _L)] for k in range(d // _L)]

    nk = d // _L
    ngrp = w // _TR

    def fill_row(wi, carry):
        for k in range(nk):
            buf_v[wi, pl.ds(k * _L, _L)] = col_v[wi, pl.ds(k * _L, _L)]
        for k in range(nk):
            buf_v[wi, pl.ds(d + k * _L, _L)] = rvecs[k]
        return carry

    handles = []
    for g in range(ngrp):
        lax.fori_loop(g * _TR, (g + 1) * _TR, fill_row, 0)
        for i in range(b):
            handles.append(
                pltpu.async_copy(
                    buf_v.at[pl.ds(g * _TR, _TR), :],
                    out_hbm.at[i, wid, pl.ds(g * _TR, _TR)],
                    sem,
                )
            )
    for hnd in handles:
        hnd.wait()


def kernel(input_tensor, row_embed, col_embed):
    b = input_tensor.shape[0]
    h, w = input_tensor.shape[-2], input_tensor.shape[-1]
    d = row_embed.shape[-1]
    mesh = plsc.VectorSubcoreMesh(core_axis_name="c", subcore_axis_name="s")
    f = pl.kernel(
        functools.partial(_sc_pos_kernel, h, w, d, b),
        out_type=jax.ShapeDtypeStruct((b, h, w, 2 * d), jnp.float32),
        mesh=mesh,
        scratch_types=[
            pltpu.VMEM((w, d), jnp.float32),
            pltpu.VMEM((_TR, d), jnp.float32),
            pltpu.VMEM((w, 2 * d), jnp.float32),
            pltpu.SemaphoreType.DMA,
            pltpu.SemaphoreType.DMA,
        ],
        compiler_params=pltpu.CompilerParams(
            needs_layout_passes=False, skip_device_barrier=True
        ),
    )
    out = f(row_embed, col_embed)
    return out.transpose(0, 3, 1, 2)


# speedup vs baseline: 1.0549x; 1.0549x over previous
import functools

import jax
import jax.numpy as jnp
from jax import lax
from jax.experimental import pallas as pl
from jax.experimental.pallas import tpu as pltpu
from jax.experimental.pallas import tpu_sc as plsc

_NC = 2
_NS = 16
_L = 16
_TR, _TC_ = 8, 128


def _sc_pos_kernel(h, w, d, b, row_hbm, col_hbm, out_hbm, col_v, rowg_v, buf_v, sem, insem):
    wid = lax.axis_index("s") * _NC + lax.axis_index("c")

    in_handles = []
    for g in range(w // _TR):
        for c in range(d // _TC_):
            in_handles.append(
                pltpu.async_copy(
                    col_hbm.at[pl.ds(g * _TR, _TR), pl.ds(c * _TC_, _TC_)],
                    col_v.at[pl.ds(g * _TR, _TR), pl.ds(c * _TC_, _TC_)],
                    insem,
                )
            )
    grp = (wid // _TR) * _TR
    for c in range(d // _TC_):
        in_handles.append(
            pltpu.async_copy(
                row_hbm.at[pl.ds(grp, _TR), pl.ds(c * _TC_, _TC_)],
                rowg_v.at[:, pl.ds(c * _TC_, _TC_)],
                insem,
            )
        )
    for hnd in in_handles:
        hnd.wait()
    rsub = wid % _TR
    rvecs = [rowg_v[rsub, pl.ds(k * _L, _L)] for k in range(d // _L)]

    nk = d // _L
    ngrp = w // _TR

    def fill_row(wi, carry):
        for k in range(nk):
            buf_v[wi, pl.ds(k * _L, _L)] = col_v[wi, pl.ds(k * _L, _L)]
        for k in range(nk):
            buf_v[wi, pl.ds(d + k * _L, _L)] = rvecs[k]
        return carry

    del ngrp
    lax.fori_loop(0, w, fill_row, 0)
    handles = [
        pltpu.async_copy(buf_v, out_hbm.at[i, wid], sem) for i in range(b)
    ]
    for hnd in handles:
        hnd.wait()


def kernel(input_tensor, row_embed, col_embed):
    b = input_tensor.shape[0]
    h, w = input_tensor.shape[-2], input_tensor.shape[-1]
    d = row_embed.shape[-1]
    mesh = plsc.VectorSubcoreMesh(core_axis_name="c", subcore_axis_name="s")
    f = pl.kernel(
        functools.partial(_sc_pos_kernel, h, w, d, b),
        out_type=jax.ShapeDtypeStruct((b, h, w, 2 * d), jnp.float32),
        mesh=mesh,
        scratch_types=[
            pltpu.VMEM((w, d), jnp.float32),
            pltpu.VMEM((_TR, d), jnp.float32),
            pltpu.VMEM((w, 2 * d), jnp.float32),
            pltpu.SemaphoreType.DMA,
            pltpu.SemaphoreType.DMA,
        ],
        compiler_params=pltpu.CompilerParams(needs_layout_passes=False),
    )
    out = f(row_embed, col_embed)
    return out.transpose(0, 3, 1, 2)


# speedup vs baseline: 1.0643x; 1.0089x over previous
import functools

import jax
import jax.numpy as jnp
from jax import lax
from jax.experimental import pallas as pl
from jax.experimental.pallas import tpu as pltpu
from jax.experimental.pallas import tpu_sc as plsc

_NC = 2
_NS = 16
_L = 16
_TR, _TC_ = 8, 128


def _sc_pos_kernel(h, w, d, b, row_hbm, col_hbm, out_hbm, rowg_v, buf_v, sem, insem):
    wid = lax.axis_index("s") * _NC + lax.axis_index("c")

    in_handles = []
    for g in range(w // _TR):
        in_handles.append(
            pltpu.async_copy(
                col_hbm.at[pl.ds(g * _TR, _TR), :],
                buf_v.at[pl.ds(g * _TR, _TR), pl.ds(0, d)],
                insem,
            )
        )
    grp = (wid // _TR) * _TR
    in_handles.append(
        pltpu.async_copy(row_hbm.at[pl.ds(grp, _TR), :], rowg_v, insem)
    )
    for hnd in in_handles:
        hnd.wait()
    rsub = wid % _TR
    rvecs = [rowg_v[rsub, pl.ds(k * _L, _L)] for k in range(d // _L)]

    nk = d // _L

    def fill_row(wi, carry):
        for k in range(nk):
            buf_v[wi, pl.ds(d + k * _L, _L)] = rvecs[k]
        return carry

    lax.fori_loop(0, w, fill_row, 0)
    handles = [
        pltpu.async_copy(buf_v, out_hbm.at[i, wid], sem) for i in range(b)
    ]
    for hnd in handles:
        hnd.wait()


def kernel(input_tensor, row_embed, col_embed):
    b = input_tensor.shape[0]
    h, w = input_tensor.shape[-2], input_tensor.shape[-1]
    d = row_embed.shape[-1]
    mesh = plsc.VectorSubcoreMesh(core_axis_name="c", subcore_axis_name="s")
    f = pl.kernel(
        functools.partial(_sc_pos_kernel, h, w, d, b),
        out_type=jax.ShapeDtypeStruct((b, h, w, 2 * d), jnp.float32),
        mesh=mesh,
        scratch_types=[
            pltpu.VMEM((_TR, d), jnp.float32),
            pltpu.VMEM((w, 2 * d), jnp.float32),
            pltpu.SemaphoreType.DMA,
            pltpu.SemaphoreType.DMA,
        ],
        compiler_params=pltpu.CompilerParams(needs_layout_passes=False),
    )
    out = f(row_embed, col_embed)
    return out.transpose(0, 3, 1, 2)
